# Initial kernel scaffold; baseline (speedup 1.0000x reference)
import jax
import jax.numpy as jnp
from jax.experimental import pallas as pl


def kernel(x, node_rep, edge_index, W1, b1, W2, b2):
    raise NotImplementedError("write your pallas kernel here")


# speedup vs baseline: 2.6167x; 2.6167x over previous
import functools

import numpy as np
import jax
import jax.numpy as jnp
from jax import lax
from jax.experimental import pallas as pl
from jax.experimental.pallas import tpu as pltpu
from jax.experimental.pallas import tpu_sc as plsc

N = 10000
E = 160000
DIM = 256
NB = 16
BD = 4
HIDDEN = 1024
TAU = 0.1
MAXDEG = 8

NPAD = 10240
EPAD = 163840
PAD_NODE = N + 16

NSUB = 16
HALF = 128
HQ = 64
ROWS_T = NPAD // NSUB
ECH = 128
ECHUNKS = (EPAD // NSUB) // ECH
UCH = 128
UCHUNKS = ROWS_T // UCH
TC_R = 256
TC_GRID = NPAD // TC_R


def _make_perms():
    P = np.zeros((BD, DIM, DIM), np.float32)
    Q = np.zeros((BD, DIM, DIM), np.float32)
    QT = np.zeros((BD, DIM, DIM), np.float32)
    for b in range(NB):
        for c in range(BD):
            for e in range(BD):
                col = b * 16 + c * 4 + e
                for d in range(BD):
                    P[d, b * 16 + d * 4 + e, col] = 1.0
                    Q[d, b * 16 + c * 4 + d, col] = 1.0
                    QT[d, b * 16 + d * 4 + c, col] = 1.0
    return P, Q, QT


_PM, _QM, _QTM = _make_perms()


def _fill_const(buf, rows, cols, value):
    val = jnp.full((16,), value, jnp.float32)

    def row(i, _):
        for q in range(cols // 16):
            buf[i, pl.ds(q * 16, 16)] = val
        return 0

    lax.fori_loop(0, rows, row, 0)


def _deg_body(src_hbm, degw_hbm, ones_v, stage_v, bcast_v, idx_v, agg_sh):
    c = lax.axis_index("c")
    s = lax.axis_index("s")
    _fill_const(ones_v, ECH, 16, 1.0)
    _fill_const(stage_v, 64, 16, 0.0)
    for m in range(ROWS_T // 64):
        pltpu.sync_copy(stage_v, agg_sh.at[pl.ds(s * ROWS_T + m * 64, 64)])
    plsc.subcore_barrier()

    ebase = s * (EPAD // NSUB)

    def chunk(j, _):
        pltpu.sync_copy(src_hbm.at[pl.ds(ebase + j * ECH, ECH)], idx_v.at[0])
        pltpu.sync_copy(ones_v, agg_sh.at[idx_v.at[0]], add=True)
        return 0

    lax.fori_loop(0, ECHUNKS, chunk, 0)
    plsc.subcore_barrier()
    wrows = NPAD // (2 * NSUB)
    wbase = (c * NSUB + s) * wrows
    for t in range(wrows // 64):
        pltpu.sync_copy(agg_sh.at[pl.ds(wbase + t * 64, 64)], stage_v)

        def brow(r, _):
            v = stage_v[r, pl.ds(0, 16)]
            for q in range(HALF // 16):
                bcast_v[r, pl.ds(q * 16, 16)] = v
            return 0

        lax.fori_loop(0, 64, brow, 0)
        pltpu.sync_copy(bcast_v, degw_hbm.at[pl.ds(wbase + t * 64, 64)])


def _diff_body(h0s_hbm, cps_hbm, rs_hbm, src_hbm, dst_hbm, hs_hbm, ps_hbm,
               zero_v, sidx_v, didx_v, gidx_v, rows_v,
               pbuf, abuf, hbuf, rbuf, agg_sh, sem):
    c = lax.axis_index("c")
    s = lax.axis_index("s")
    trow = s * ROWS_T
    ebase = s * (EPAD // NSUB)

    _fill_const(zero_v, UCH, HQ, 0.0)

    def quarter(q, _):
        qbase = (2 * c + q) * NPAD

        for m in range(UCHUNKS):
            r0 = qbase + trow + m * UCH
            pltpu.sync_copy(h0s_hbm.at[pl.ds(r0, UCH)], hbuf)
            pltpu.sync_copy(hbuf, hs_hbm.at[pl.ds(r0, UCH)])
            pltpu.sync_copy(cps_hbm.at[pl.ds(r0, UCH)], pbuf)
            pltpu.sync_copy(pbuf, ps_hbm.at[pl.ds(r0, UCH)])
        plsc.subcore_barrier()

        def round_body(coef):
            for m in range(UCHUNKS):
                pltpu.sync_copy(zero_v, agg_sh.at[pl.ds(trow + m * UCH, UCH)])
            plsc.subcore_barrier()

            def chunk(j, _):
                eb = ebase + j * ECH
                pltpu.sync_copy(src_hbm.at[pl.ds(eb, ECH)], sidx_v.at[0])
                pltpu.sync_copy(dst_hbm.at[pl.ds(eb, ECH)], didx_v.at[0])
                for w in range(ECH // 16):
                    sl = pl.ds(w * 16, 16)
                    gidx_v[0, sl] = sidx_v[0, sl] + qbase
                pltpu.async_copy(ps_hbm.at[gidx_v.at[0]], rows_v, sem).wait()
                pltpu.sync_copy(rows_v, agg_sh.at[didx_v.at[0]], add=True)
                return 0

            lax.fori_loop(0, ECHUNKS, chunk, 0)
            plsc.subcore_barrier()

            def upd(m, _):
                r0 = trow + m * UCH
                pltpu.sync_copy(ps_hbm.at[pl.ds(qbase + r0, UCH)], pbuf)
                pltpu.sync_copy(agg_sh.at[pl.ds(r0, UCH)], abuf)
                pltpu.sync_copy(hs_hbm.at[pl.ds(qbase + r0, UCH)], hbuf)
                pltpu.sync_copy(rs_hbm.at[pl.ds(r0, UCH)], rbuf)

                def urow(r, _):
                    for w in range(HQ // 16):
                        sl = pl.ds(w * 16, 16)
                        p = pbuf[r, sl]
                        rs = rbuf[r, sl]
                        a = abuf[r, sl]
                        cn = coef * (p / rs - rs * a)
                        hbuf[r, sl] = hbuf[r, sl] + cn
                        pbuf[r, sl] = rs * cn
                    return 0

                lax.fori_loop(0, UCH, urow, 0)
                pltpu.sync_copy(pbuf, ps_hbm.at[pl.ds(qbase + r0, UCH)])
                pltpu.sync_copy(hbuf, hs_hbm.at[pl.ds(qbase + r0, UCH)])
                return 0

            lax.fori_loop(0, UCHUNKS, upd, 0)
            plsc.subcore_barrier()

        for k in range(1, MAXDEG + 1):
            round_body(-TAU / k)
        return 0

    lax.fori_loop(0, 2, quarter, 0)


@functools.lru_cache(maxsize=1)
def _sc_kernels():
    mesh = plsc.VectorSubcoreMesh(core_axis_name="c", subcore_axis_name="s",
                                  num_cores=2, num_subcores=NSUB)
    deg = pl.kernel(
        _deg_body,
        out_type=jax.ShapeDtypeStruct((NPAD, HALF), jnp.float32),
        mesh=mesh,
        scratch_types=[
            pltpu.VMEM((ECH, 16), jnp.float32),
            pltpu.VMEM((64, 16), jnp.float32),
            pltpu.VMEM((64, HALF), jnp.float32),
            pltpu.VMEM((1, ECH), jnp.int32),
            pltpu.VMEM_SHARED((NPAD, 16), jnp.float32),
        ],
        compiler_params=pltpu.CompilerParams(use_tc_tiling_on_sc=False),
    )
    diff = pl.kernel(
        _diff_body,
        out_type=(
            jax.ShapeDtypeStruct((4 * NPAD, HQ), jnp.float32),
            jax.ShapeDtypeStruct((4 * NPAD, HQ), jnp.float32),
        ),
        mesh=mesh,
        scratch_types=[
            pltpu.VMEM((UCH, HQ), jnp.float32),
            pltpu.VMEM((1, ECH), jnp.int32),
            pltpu.VMEM((1, ECH), jnp.int32),
            pltpu.VMEM((1, ECH), jnp.int32),
            pltpu.VMEM((ECH, HQ), jnp.float32),
            pltpu.VMEM((UCH, HQ), jnp.float32),
            pltpu.VMEM((UCH, HQ), jnp.float32),
            pltpu.VMEM((UCH, HQ), jnp.float32),
            pltpu.VMEM((UCH, HQ), jnp.float32),
            pltpu.VMEM_SHARED((NPAD, HQ), jnp.float32),
            pltpu.SemaphoreType.DMA,
        ],
        compiler_params=pltpu.CompilerParams(use_tc_tiling_on_sc=False),
    )
    return deg, diff


def _tca_body(x_ref, rep_ref, deg_ref, p_ref, q_ref,
              h0a_ref, h0b_ref, cpa_ref, cpb_ref, rs_ref):
    xb = x_ref[...]
    rb = rep_ref[...]
    h0 = jnp.zeros_like(xb)
    for d in range(BD):
        xd = jnp.dot(xb, p_ref[d], preferred_element_type=jnp.float32)
        rd = jnp.dot(rb, q_ref[d], preferred_element_type=jnp.float32)
        h0 = h0 + rd * xd
    rs = lax.rsqrt(jnp.maximum(deg_ref[...], 1.0))
    cp = h0 * rs[:, 0:1]
    h0a_ref[...] = h0[:, :HALF]
    h0b_ref[...] = h0[:, HALF:]
    cpa_ref[...] = cp[:, :HALF]
    cpb_ref[...] = cp[:, HALF:]
    rs_ref[...] = rs


def _tca_call(xp, repf, degw):
    row = lambda i: (i, 0)
    out128 = jax.ShapeDtypeStruct((NPAD, HALF), jnp.float32)
    return pl.pallas_call(
        _tca_body,
        grid=(TC_GRID,),
        in_specs=[
            pl.BlockSpec((TC_R, DIM), row),
            pl.BlockSpec((TC_R, DIM), row),
            pl.BlockSpec((TC_R, HALF), row),
            pl.BlockSpec((BD, DIM, DIM), lambda i: (0, 0, 0)),
            pl.BlockSpec((BD, DIM, DIM), lambda i: (0, 0, 0)),
        ],
        out_specs=[
            pl.BlockSpec((TC_R, HALF), row),
            pl.BlockSpec((TC_R, HALF), row),
            pl.BlockSpec((TC_R, HALF), row),
            pl.BlockSpec((TC_R, HALF), row),
            pl.BlockSpec((TC_R, HALF), row),
        ],
        out_shape=[out128, out128, out128, out128, out128],
    )(xp, repf, degw, _PM, _QM)


def _tcb_body(x_ref, h_ref, rep_ref, w1x_ref, w1h_ref, b1_ref, w2_ref, b2_ref,
              p_ref, qt_ref, out_ref):
    xb = x_ref[...]
    hb = h_ref[...]
    y1 = (jnp.dot(xb, w1x_ref[...], preferred_element_type=jnp.float32)
          + jnp.dot(hb, w1h_ref[...], preferred_element_type=jnp.float32)
          + b1_ref[0:1, :])
    g = jax.nn.gelu(y1)
    y2 = jnp.dot(g, w2_ref[...], preferred_element_type=jnp.float32) + b2_ref[0:1, :]
    rb = rep_ref[...]
    out = jnp.zeros_like(y2)
    for d in range(BD):
        td = jnp.dot(rb, qt_ref[d], preferred_element_type=jnp.float32)
        yd = jnp.dot(y2, p_ref[d], preferred_element_type=jnp.float32)
        out = out + td * yd
    out_ref[...] = out


def _tcb_call(xp, h, repf, w1x, w1h, b1p, w2, b2p):
    row = lambda i: (i, 0)
    fixed = lambda i: (0, 0)
    return pl.pallas_call(
        _tcb_body,
        grid=(TC_GRID,),
        in_specs=[
            pl.BlockSpec((TC_R, DIM), row),
            pl.BlockSpec((TC_R, DIM), row),
            pl.BlockSpec((TC_R, DIM), row),
            pl.BlockSpec((DIM, HIDDEN), fixed),
            pl.BlockSpec((DIM, HIDDEN), fixed),
            pl.BlockSpec((8, HIDDEN), fixed),
            pl.BlockSpec((HIDDEN, DIM), fixed),
            pl.BlockSpec((8, DIM), fixed),
            pl.BlockSpec((BD, DIM, DIM), lambda i: (0, 0, 0)),
            pl.BlockSpec((BD, DIM, DIM), lambda i: (0, 0, 0)),
        ],
        out_specs=pl.BlockSpec((TC_R, DIM), row),
        out_shape=jax.ShapeDtypeStruct((NPAD, DIM), jnp.float32),
    )(xp, h, repf, w1x, w1h, b1p, w2, b2p, _PM, _QTM)


def kernel(x, node_rep, edge_index, W1, b1, W2, b2):
    xp = jnp.pad(x, ((0, NPAD - N), (0, 0)))
    repf = jnp.pad(node_rep.reshape(N, DIM), ((0, NPAD - N), (0, 0)))
    srcp = jnp.pad(edge_index[0].astype(jnp.int32), (0, EPAD - E),
                   constant_values=PAD_NODE)
    dstp = jnp.pad(edge_index[1].astype(jnp.int32), (0, EPAD - E),
                   constant_values=PAD_NODE)

    deg_kernel, diff_kernel = _sc_kernels()
    degw = deg_kernel(srcp)
    h0a, h0b, cpa, cpb, rsb = _tca_call(xp, repf, degw)
    h0s = jnp.concatenate([h0a[:, :HQ], h0a[:, HQ:], h0b[:, :HQ], h0b[:, HQ:]],
                          axis=0)
    cps = jnp.concatenate([cpa[:, :HQ], cpa[:, HQ:], cpb[:, :HQ], cpb[:, HQ:]],
                          axis=0)
    hs, _ = diff_kernel(h0s, cps, rsb[:, :HQ], srcp, dstp)
    h = jnp.concatenate([hs[0:NPAD], hs[NPAD:2 * NPAD],
                         hs[2 * NPAD:3 * NPAD], hs[3 * NPAD:]], axis=1)

    b1p = jnp.pad(b1[None, :], ((0, 7), (0, 0)))
    b2p = jnp.pad(b2[None, :], ((0, 7), (0, 0)))
    out = _tcb_call(xp, h, repf, W1[:DIM], W1[DIM:], b1p, W2, b2p)
    return out[:N]


# speedup vs baseline: 3.9645x; 1.5151x over previous
import functools

import numpy as np
import jax
import jax.numpy as jnp
from jax import lax
from jax.experimental import pallas as pl
from jax.experimental.pallas import tpu as pltpu
from jax.experimental.pallas import tpu_sc as plsc

N = 10000
E = 160000
DIM = 256
NB = 16
BD = 4
HIDDEN = 1024
TAU = 0.1
MAXDEG = 8

NPAD = 10240
EPAD = 163840
PAD_NODE = N + 16

NSUB = 16
HALF = 128
HQ = 64
ROWS_T = NPAD // NSUB
ECH = 128
ECHUNKS = (EPAD // NSUB) // ECH
UCH = 128
UCHUNKS = ROWS_T // UCH
TC_R = 256
TC_GRID = NPAD // TC_R


def _make_perms():
    P = np.zeros((BD, DIM, DIM), np.float32)
    Q = np.zeros((BD, DIM, DIM), np.float32)
    QT = np.zeros((BD, DIM, DIM), np.float32)
    for b in range(NB):
        for c in range(BD):
            for e in range(BD):
                col = b * 16 + c * 4 + e
                for d in range(BD):
                    P[d, b * 16 + d * 4 + e, col] = 1.0
                    Q[d, b * 16 + c * 4 + d, col] = 1.0
                    QT[d, b * 16 + d * 4 + c, col] = 1.0
    return P, Q, QT


_PM, _QM, _QTM = _make_perms()


def _fill_const(buf, rows, cols, value):
    val = jnp.full((16,), value, jnp.float32)

    def row(i, _):
        for q in range(cols // 16):
            buf[i, pl.ds(q * 16, 16)] = val
        return 0

    lax.fori_loop(0, rows, row, 0)


def _deg_body(src_hbm, degw_hbm, ones_v, stage_v, bcast_v, idx_v, agg_sh):
    c = lax.axis_index("c")
    s = lax.axis_index("s")
    _fill_const(ones_v, ECH, 16, 1.0)
    _fill_const(stage_v, 64, 16, 0.0)
    for m in range(ROWS_T // 64):
        pltpu.sync_copy(stage_v, agg_sh.at[pl.ds(s * ROWS_T + m * 64, 64)])
    plsc.subcore_barrier()

    ebase = s * (EPAD // NSUB)

    def chunk(j, _):
        pltpu.sync_copy(src_hbm.at[pl.ds(ebase + j * ECH, ECH)], idx_v.at[0])
        pltpu.sync_copy(ones_v, agg_sh.at[idx_v.at[0]], add=True)
        return 0

    lax.fori_loop(0, ECHUNKS, chunk, 0)
    plsc.subcore_barrier()
    wrows = NPAD // (2 * NSUB)
    wbase = (c * NSUB + s) * wrows
    for t in range(wrows // 64):
        pltpu.sync_copy(agg_sh.at[pl.ds(wbase + t * 64, 64)], stage_v)

        def brow(r, _):
            v = stage_v[r, pl.ds(0, 16)]
            for q in range(HALF // 16):
                bcast_v[r, pl.ds(q * 16, 16)] = v
            return 0

        lax.fori_loop(0, 64, brow, 0)
        pltpu.sync_copy(bcast_v, degw_hbm.at[pl.ds(wbase + t * 64, 64)])


UCH2 = 64


def _diff_body(h0s_hbm, cps_hbm, rs2_hbm, ri_hbm, src_hbm, dst_hbm,
               hs_hbm, ps_hbm,
               zero_v, gidx_v, didx_v, rows_v, pbuf, hbuf, agg_v,
               rs2_v, ri_v, agg_sh, gsem, ssem):
    c = lax.axis_index("c")
    s = lax.axis_index("s")
    trow = s * ROWS_T
    ebase = s * (EPAD // NSUB)

    _fill_const(zero_v, UCH2, HQ, 0.0)

    def ldidx(t, _):
        eb = ebase + t * ECH
        pltpu.sync_copy(src_hbm.at[pl.ds(eb, ECH)], gidx_v.at[t])
        pltpu.sync_copy(dst_hbm.at[pl.ds(eb, ECH)], didx_v.at[t])
        return 0

    lax.fori_loop(0, ECHUNKS, ldidx, 0)
    pltpu.sync_copy(rs2_hbm.at[pl.ds(trow, ROWS_T)], rs2_v)
    pltpu.sync_copy(ri_hbm.at[pl.ds(trow, ROWS_T)], ri_v)

    def quarter(q, _):
        qbase = (2 * c + q) * NPAD
        delta = jnp.where(q == 0, 2 * c * NPAD, NPAD).astype(jnp.int32)

        def shift(t, _):
            for w in range(ECH // 16):
                sl = pl.ds(w * 16, 16)
                gidx_v[t, sl] = gidx_v[t, sl] + delta
            return 0

        lax.fori_loop(0, ECHUNKS, shift, 0)

        for m in range(ROWS_T // UCH2):
            r0 = qbase + trow + m * UCH2
            pltpu.sync_copy(h0s_hbm.at[pl.ds(r0, UCH2)], hbuf)
            pltpu.sync_copy(hbuf, hs_hbm.at[pl.ds(r0, UCH2)])
            pltpu.sync_copy(cps_hbm.at[pl.ds(r0, UCH2)], pbuf)
            pltpu.sync_copy(pbuf, ps_hbm.at[pl.ds(r0, UCH2)])
        plsc.subcore_barrier()

        def round_body(coef):
            for m in range(ROWS_T // UCH2):
                pltpu.sync_copy(zero_v, agg_sh.at[pl.ds(trow + m * UCH2, UCH2)])
            plsc.subcore_barrier()

            pltpu.async_copy(ps_hbm.at[gidx_v.at[0]], rows_v.at[0], gsem.at[0])

            def chunk(j, _):
                b = lax.rem(j, 2)
                nb = 1 - b

                @pl.when(j + 1 < ECHUNKS)
                def _():
                    @pl.when(j >= 1)
                    def _():
                        pltpu.make_async_copy(rows_v.at[nb],
                                              agg_sh.at[didx_v.at[j - 1]],
                                              ssem.at[nb]).wait()

                    pltpu.async_copy(ps_hbm.at[gidx_v.at[j + 1]],
                                     rows_v.at[nb], gsem.at[nb])

                pltpu.make_async_copy(ps_hbm.at[gidx_v.at[j]],
                                      rows_v.at[b], gsem.at[b]).wait()
                pltpu.async_copy(rows_v.at[b], agg_sh.at[didx_v.at[j]],
                                 ssem.at[b], add=True)
                return 0

            lax.fori_loop(0, ECHUNKS, chunk, 0)
            pltpu.make_async_copy(rows_v.at[(ECHUNKS - 2) % 2],
                                  agg_sh.at[didx_v.at[ECHUNKS - 2]],
                                  ssem.at[(ECHUNKS - 2) % 2]).wait()
            pltpu.make_async_copy(rows_v.at[(ECHUNKS - 1) % 2],
                                  agg_sh.at[didx_v.at[ECHUNKS - 1]],
                                  ssem.at[(ECHUNKS - 1) % 2]).wait()
            plsc.subcore_barrier()

            def upd(m, _):
                r0 = trow + m * UCH2
                pltpu.sync_copy(agg_sh.at[pl.ds(r0, UCH2)], agg_v)
                pltpu.sync_copy(ps_hbm.at[pl.ds(qbase + r0, UCH2)], pbuf)
                pltpu.sync_copy(hs_hbm.at[pl.ds(qbase + r0, UCH2)], hbuf)

                def ugroup(g, _):
                    base = m * UCH2 + g * 16
                    rs2_16 = rs2_v[pl.ds(base, 16)]
                    ri_16 = ri_v[pl.ds(base, 16)]
                    for li in range(16):
                        r = g * 16 + li
                        rs2 = jnp.broadcast_to(rs2_16[li], (16,))
                        ri = jnp.broadcast_to(ri_16[li], (16,))
                        for w in range(HQ // 16):
                            sl = pl.ds(w * 16, 16)
                            pn = coef * (pbuf[r, sl] - rs2 * agg_v[r, sl])
                            pbuf[r, sl] = pn
                            hbuf[r, sl] = hbuf[r, sl] + pn * ri
                    return 0

                lax.fori_loop(0, UCH2 // 16, ugroup, 0)
                pltpu.sync_copy(pbuf, ps_hbm.at[pl.ds(qbase + r0, UCH2)])
                pltpu.sync_copy(hbuf, hs_hbm.at[pl.ds(qbase + r0, UCH2)])
                return 0

            lax.fori_loop(0, ROWS_T // UCH2, upd, 0)
            plsc.subcore_barrier()

        for k in range(1, MAXDEG + 1):
            round_body(-TAU / k)
        return 0

    lax.fori_loop(0, 2, quarter, 0)


@functools.lru_cache(maxsize=1)
def _sc_kernels():
    mesh = plsc.VectorSubcoreMesh(core_axis_name="c", subcore_axis_name="s",
                                  num_cores=2, num_subcores=NSUB)
    deg = pl.kernel(
        _deg_body,
        out_type=jax.ShapeDtypeStruct((NPAD, HALF), jnp.float32),
        mesh=mesh,
        scratch_types=[
            pltpu.VMEM((ECH, 16), jnp.float32),
            pltpu.VMEM((64, 16), jnp.float32),
            pltpu.VMEM((64, HALF), jnp.float32),
            pltpu.VMEM((1, ECH), jnp.int32),
            pltpu.VMEM_SHARED((NPAD, 16), jnp.float32),
        ],
        compiler_params=pltpu.CompilerParams(use_tc_tiling_on_sc=False),
    )
    diff = pl.kernel(
        _diff_body,
        out_type=(
            jax.ShapeDtypeStruct((4 * NPAD, HQ), jnp.float32),
            jax.ShapeDtypeStruct((4 * NPAD, HQ), jnp.float32),
        ),
        mesh=mesh,
        scratch_types=[
            pltpu.VMEM((UCH2, HQ), jnp.float32),
            pltpu.VMEM((ECHUNKS, ECH), jnp.int32),
            pltpu.VMEM((ECHUNKS, ECH), jnp.int32),
            pltpu.VMEM((2, ECH, HQ), jnp.float32),
            pltpu.VMEM((UCH2, HQ), jnp.float32),
            pltpu.VMEM((UCH2, HQ), jnp.float32),
            pltpu.VMEM((UCH2, HQ), jnp.float32),
            pltpu.VMEM((ROWS_T,), jnp.float32),
            pltpu.VMEM((ROWS_T,), jnp.float32),
            pltpu.VMEM_SHARED((NPAD, HQ), jnp.float32),
            pltpu.SemaphoreType.DMA((2,)),
            pltpu.SemaphoreType.DMA((2,)),
        ],
        compiler_params=pltpu.CompilerParams(use_tc_tiling_on_sc=False),
    )
    return deg, diff


def _tca_body(x_ref, rep_ref, deg_ref, p_ref, q_ref,
              h0a_ref, h0b_ref, cpa_ref, cpb_ref, rs2_ref, ri_ref):
    xb = x_ref[...]
    rb = rep_ref[...]
    h0 = jnp.zeros_like(xb)
    for d in range(BD):
        xd = jnp.dot(xb, p_ref[d], preferred_element_type=jnp.float32)
        rd = jnp.dot(rb, q_ref[d], preferred_element_type=jnp.float32)
        h0 = h0 + rd * xd
    deg = jnp.maximum(deg_ref[...], 1.0)
    rs = lax.rsqrt(deg)
    cp = h0 * rs[:, 0:1]
    h0a_ref[...] = h0[:, :HALF]
    h0b_ref[...] = h0[:, HALF:]
    cpa_ref[...] = cp[:, :HALF]
    cpb_ref[...] = cp[:, HALF:]
    rs2_ref[...] = 1.0 / deg
    ri_ref[...] = jnp.sqrt(deg)


def _tca_call(xp, repf, degw):
    row = lambda i: (i, 0)
    out128 = jax.ShapeDtypeStruct((NPAD, HALF), jnp.float32)
    return pl.pallas_call(
        _tca_body,
        grid=(TC_GRID,),
        in_specs=[
            pl.BlockSpec((TC_R, DIM), row),
            pl.BlockSpec((TC_R, DIM), row),
            pl.BlockSpec((TC_R, HALF), row),
            pl.BlockSpec((BD, DIM, DIM), lambda i: (0, 0, 0)),
            pl.BlockSpec((BD, DIM, DIM), lambda i: (0, 0, 0)),
        ],
        out_specs=[
            pl.BlockSpec((TC_R, HALF), row),
            pl.BlockSpec((TC_R, HALF), row),
            pl.BlockSpec((TC_R, HALF), row),
            pl.BlockSpec((TC_R, HALF), row),
            pl.BlockSpec((TC_R, HALF), row),
            pl.BlockSpec((TC_R, HALF), row),
        ],
        out_shape=[out128, out128, out128, out128, out128, out128],
    )(xp, repf, degw, _PM, _QM)


def _tcb_body(x_ref, h_ref, rep_ref, w1x_ref, w1h_ref, b1_ref, w2_ref, b2_ref,
              p_ref, qt_ref, out_ref):
    xb = x_ref[...]
    hb = h_ref[...]
    y1 = (jnp.dot(xb, w1x_ref[...], preferred_element_type=jnp.float32)
          + jnp.dot(hb, w1h_ref[...], preferred_element_type=jnp.float32)
          + b1_ref[0:1, :])
    g = jax.nn.gelu(y1)
    y2 = jnp.dot(g, w2_ref[...], preferred_element_type=jnp.float32) + b2_ref[0:1, :]
    rb = rep_ref[...]
    out = jnp.zeros_like(y2)
    for d in range(BD):
        td = jnp.dot(rb, qt_ref[d], preferred_element_type=jnp.float32)
        yd = jnp.dot(y2, p_ref[d], preferred_element_type=jnp.float32)
        out = out + td * yd
    out_ref[...] = out


def _tcb_call(xp, h, repf, w1x, w1h, b1p, w2, b2p):
    row = lambda i: (i, 0)
    fixed = lambda i: (0, 0)
    return pl.pallas_call(
        _tcb_body,
        grid=(TC_GRID,),
        in_specs=[
            pl.BlockSpec((TC_R, DIM), row),
            pl.BlockSpec((TC_R, DIM), row),
            pl.BlockSpec((TC_R, DIM), row),
            pl.BlockSpec((DIM, HIDDEN), fixed),
            pl.BlockSpec((DIM, HIDDEN), fixed),
            pl.BlockSpec((8, HIDDEN), fixed),
            pl.BlockSpec((HIDDEN, DIM), fixed),
            pl.BlockSpec((8, DIM), fixed),
            pl.BlockSpec((BD, DIM, DIM), lambda i: (0, 0, 0)),
            pl.BlockSpec((BD, DIM, DIM), lambda i: (0, 0, 0)),
        ],
        out_specs=pl.BlockSpec((TC_R, DIM), row),
        out_shape=jax.ShapeDtypeStruct((NPAD, DIM), jnp.float32),
    )(xp, h, repf, w1x, w1h, b1p, w2, b2p, _PM, _QTM)


def kernel(x, node_rep, edge_index, W1, b1, W2, b2):
    xp = jnp.pad(x, ((0, NPAD - N), (0, 0)))
    repf = jnp.pad(node_rep.reshape(N, DIM), ((0, NPAD - N), (0, 0)))
    srcp = jnp.pad(edge_index[0].astype(jnp.int32), (0, EPAD - E),
                   constant_values=PAD_NODE)
    dstp = jnp.pad(edge_index[1].astype(jnp.int32), (0, EPAD - E),
                   constant_values=PAD_NODE)

    deg_kernel, diff_kernel = _sc_kernels()
    degw = deg_kernel(srcp)
    h0a, h0b, cpa, cpb, rs2b, rib = _tca_call(xp, repf, degw)
    h0s = jnp.concatenate([h0a[:, :HQ], h0a[:, HQ:], h0b[:, :HQ], h0b[:, HQ:]],
                          axis=0)
    cps = jnp.concatenate([cpa[:, :HQ], cpa[:, HQ:], cpb[:, :HQ], cpb[:, HQ:]],
                          axis=0)
    hs, _ = diff_kernel(h0s, cps, rs2b[:, 0], rib[:, 0], srcp, dstp)
    h = jnp.concatenate([hs[0:NPAD], hs[NPAD:2 * NPAD],
                         hs[2 * NPAD:3 * NPAD], hs[3 * NPAD:]], axis=1)

    b1p = jnp.pad(b1[None, :], ((0, 7), (0, 0)))
    b2p = jnp.pad(b2[None, :], ((0, 7), (0, 0)))
    out = _tcb_call(xp, h, repf, W1[:DIM], W1[DIM:], b1p, W2, b2p)
    return out[:N]
